# baseline (speedup 1.0000x reference)
import functools

import jax
import jax.numpy as jnp
from jax import lax
from jax.experimental import pallas as pl
from jax.experimental.pallas import tpu as pltpu
from jax.experimental.pallas import tpu_sc as plsc

_NUM_FIELDS = 26
_FIELD_DIM = 38462
_VOCAB = _NUM_FIELDS * _FIELD_DIM
_EMBED_DIM = 16
_BATCH = 4096
_NUM_WORKERS = 32
_BPW = _BATCH // _NUM_WORKERS
_LANES = 16

_CONV_COLS = 512
_CONV_GRID = -(-_VOCAB // _CONV_COLS)
_CONV_ROWS = _CONV_GRID * (_CONV_COLS // 8)


def _conv_body(x_ref, o_ref):
    for u in range(8):
        o_ref[:, 16 * u:16 * u + 16] = x_ref[:, 64 * u:64 * u + 64].T


@jax.jit
def _convert_table(table_t):
    return pl.pallas_call(
        _conv_body,
        grid=(_CONV_GRID,),
        in_specs=[pl.BlockSpec((_EMBED_DIM, _CONV_COLS), lambda i: (0, i))],
        out_specs=pl.BlockSpec((_CONV_COLS // 8, 128), lambda i: (i, 0)),
        out_shape=jax.ShapeDtypeStruct((_CONV_ROWS, 128), jnp.float32),
    )(table_t)


def _gather_body(idx_hbm, offs_hbm, tab_hbm, out_hbm,
                 idx_v, offs_v, row_v, cb_v, buf0, buf1, slab_v,
                 sem0, sem1):
    wid = lax.axis_index("s") * 2 + lax.axis_index("c")
    base = wid * _BPW

    pltpu.sync_copy(idx_hbm.at[:, pl.ds(base, _BPW)], idx_v)
    pltpu.sync_copy(offs_hbm, offs_v)

    def adjust(f, carry):
        fvec = jnp.full((_LANES,), f, dtype=jnp.int32)
        off = plsc.load_gather(offs_v, [fvec])
        for j in range(_BPW // _LANES):
            sl = pl.ds(j * _LANES, _LANES)
            v = idx_v[f, sl] + off
            row_v[f, sl] = lax.shift_left(
                lax.shift_right_logical(v, 9), 6
            ) + jnp.bitwise_and(v, 63)
            cb_v[f, sl] = lax.shift_left(
                jnp.bitwise_and(lax.shift_right_logical(v, 6), 7), 4
            )
        return carry

    lax.fori_loop(0, _NUM_FIELDS, adjust, 0)

    def fire(f, buf, sem):
        pltpu.async_copy(tab_hbm.at[row_v.at[f]], buf, sem)

    def drain(f, buf, sem):
        pltpu.make_async_copy(tab_hbm.at[row_v.at[f]], buf, sem).wait()

    def extract(f, buf):
        for j in range(_BPW // _LANES):
            rows = jnp.arange(_LANES, dtype=jnp.int32) + (j * _LANES)
            cb = cb_v[f, pl.ds(j * _LANES, _LANES)]
            for e in range(_EMBED_DIM):
                vals = plsc.load_gather(buf, [rows, cb + e])
                slab_v[f, e // 8, e % 8, pl.ds(j * _LANES, _LANES)] = vals

    fire(0, buf0, sem0)
    fire(1, buf1, sem1)

    def step(g, carry):
        f = g * 2
        drain(f, buf0, sem0)
        extract(f, buf0)

        @pl.when(f + 2 < _NUM_FIELDS)
        def _():
            fire(f + 2, buf0, sem0)

        drain(f + 1, buf1, sem1)
        extract(f + 1, buf1)

        @pl.when(f + 3 < _NUM_FIELDS)
        def _():
            fire(f + 3, buf1, sem1)

        return carry

    lax.fori_loop(0, _NUM_FIELDS // 2, step, 0)

    pltpu.sync_copy(slab_v, out_hbm.at[:, :, wid])


@jax.jit
def _fmembedding(idx_t, offsets, tab):
    mesh = plsc.VectorSubcoreMesh(
        core_axis_name="c", subcore_axis_name="s", num_cores=2, num_subcores=16
    )
    run = functools.partial(
        pl.kernel,
        out_type=jax.ShapeDtypeStruct(
            (_NUM_FIELDS, 2, _NUM_WORKERS, 8, _BPW), jnp.float32
        ),
        mesh=mesh,
        scratch_types=[
            pltpu.VMEM((_NUM_FIELDS, _BPW), jnp.int32),
            pltpu.VMEM((_NUM_FIELDS,), jnp.int32),
            pltpu.VMEM((_NUM_FIELDS, _BPW), jnp.int32),
            pltpu.VMEM((_NUM_FIELDS, _BPW), jnp.int32),
            pltpu.VMEM((_BPW, 128), jnp.float32),
            pltpu.VMEM((_BPW, 128), jnp.float32),
            pltpu.VMEM((_NUM_FIELDS, 2, 8, _BPW), jnp.float32),
            pltpu.SemaphoreType.DMA,
            pltpu.SemaphoreType.DMA,
        ],
        compiler_params=pltpu.CompilerParams(
            use_tc_tiling_on_sc=True, needs_layout_passes=False
        ),
    )(_gather_body)
    return run(idx_t, offsets, tab)


def kernel(input_x, table, offsets):
    tab = _convert_table(table.T)
    out5 = _fmembedding(input_x.T, offsets, tab)
    return out5.transpose(2, 4, 0, 1, 3).reshape(_BATCH, _NUM_FIELDS, _EMBED_DIM)


# speedup vs baseline: 4.7715x; 4.7715x over previous
import functools

import jax
import jax.numpy as jnp
import numpy as np
from jax import lax
from jax.experimental import pallas as pl
from jax.experimental.pallas import tpu as pltpu
from jax.experimental.pallas import tpu_sc as plsc

_NUM_FIELDS = 26
_FIELD_DIM = 38462
_VOCAB = _NUM_FIELDS * _FIELD_DIM
_EMBED_DIM = 16
_BATCH = 4096
_NUM_WORKERS = 32
_BPW = _BATCH // _NUM_WORKERS
_LANES = 16

_CONV_COLS = 4096
_CONV_GRID = -(-_VOCAB // _CONV_COLS)
_CONV_ROWS = _CONV_GRID * (_CONV_COLS // 8)


_SEL = np.zeros((8, 16, 128), np.float32)
for _u in range(8):
    for _e in range(16):
        _SEL[_u, _e, 16 * _u + _e] = 1.0


def _conv_body(x_ref, sel_ref, o_ref):
    x = x_ref[...]
    sel = sel_ref[...]
    for s in range(_CONV_COLS // 1024):
        xs = x[:, 1024 * s:1024 * s + 1024]
        acc = None
        for u in range(8):
            d = lax.dot_general(
                xs[:, 128 * u:128 * u + 128],
                sel[u],
                (((0,), (0,)), ((), ())),
                preferred_element_type=jnp.float32,
            )
            acc = d if acc is None else acc + d
        o_ref[pl.ds(128 * s, 128), :] = acc


@jax.jit
def _convert_table(table_t):
    return pl.pallas_call(
        _conv_body,
        grid=(_CONV_GRID,),
        in_specs=[
            pl.BlockSpec((_EMBED_DIM, _CONV_COLS), lambda i: (0, i)),
            pl.BlockSpec((8, 16, 128), lambda i: (0, 0, 0)),
        ],
        out_specs=pl.BlockSpec((_CONV_COLS // 8, 128), lambda i: (i, 0)),
        compiler_params=pltpu.CompilerParams(
            dimension_semantics=("arbitrary",),
            fuse_transposed_lhs_in_matmul=True,
        ),
        out_shape=jax.ShapeDtypeStruct((_CONV_ROWS, 128), jnp.float32),
    )(table_t, jnp.asarray(_SEL))


def _gather_body(idx_hbm, offs_hbm, tab_hbm, out_hbm,
                 idx_v, offs_v, row_v, cb_v, buf0, buf1, slab_v,
                 sem0, sem1):
    wid = lax.axis_index("s") * 2 + lax.axis_index("c")
    base = wid * _BPW

    pltpu.sync_copy(idx_hbm.at[:, pl.ds(base, _BPW)], idx_v)
    pltpu.sync_copy(offs_hbm, offs_v)

    def adjust(f, carry):
        fvec = jnp.full((_LANES,), f, dtype=jnp.int32)
        off = plsc.load_gather(offs_v, [fvec])
        for j in range(_BPW // _LANES):
            sl = pl.ds(j * _LANES, _LANES)
            v = idx_v[f, sl] + off
            row_v[f, sl] = lax.shift_left(
                lax.shift_right_logical(v, 10), 7
            ) + jnp.bitwise_and(v, 127)
            cb_v[f, sl] = lax.shift_left(
                jnp.bitwise_and(lax.shift_right_logical(v, 7), 7), 4
            )
        return carry

    lax.fori_loop(0, _NUM_FIELDS, adjust, 0)

    def fire(f, buf, sem):
        pltpu.async_copy(tab_hbm.at[row_v.at[f]], buf, sem)

    def drain(f, buf, sem):
        pltpu.make_async_copy(tab_hbm.at[row_v.at[f]], buf, sem).wait()

    def extract(f, buf):
        for j in range(_BPW // _LANES):
            rows = jnp.arange(_LANES, dtype=jnp.int32) + (j * _LANES)
            cb = cb_v[f, pl.ds(j * _LANES, _LANES)]
            for e in range(_EMBED_DIM):
                vals = plsc.load_gather(buf, [rows, cb + e])
                slab_v[f, e // 8, e % 8, pl.ds(j * _LANES, _LANES)] = vals

    fire(0, buf0, sem0)
    fire(1, buf1, sem1)

    def step(g, carry):
        f = g * 2
        drain(f, buf0, sem0)
        extract(f, buf0)

        @pl.when(f + 2 < _NUM_FIELDS)
        def _():
            fire(f + 2, buf0, sem0)

        drain(f + 1, buf1, sem1)
        extract(f + 1, buf1)

        @pl.when(f + 3 < _NUM_FIELDS)
        def _():
            fire(f + 3, buf1, sem1)

        return carry

    lax.fori_loop(0, _NUM_FIELDS // 2, step, 0)

    pltpu.sync_copy(slab_v, out_hbm.at[:, :, wid])


@jax.jit
def _fmembedding(idx_t, offsets, tab):
    mesh = plsc.VectorSubcoreMesh(
        core_axis_name="c", subcore_axis_name="s", num_cores=2, num_subcores=16
    )
    run = functools.partial(
        pl.kernel,
        out_type=jax.ShapeDtypeStruct(
            (_NUM_FIELDS, 2, _NUM_WORKERS, 8, _BPW), jnp.float32
        ),
        mesh=mesh,
        scratch_types=[
            pltpu.VMEM((_NUM_FIELDS, _BPW), jnp.int32),
            pltpu.VMEM((_NUM_FIELDS,), jnp.int32),
            pltpu.VMEM((_NUM_FIELDS, _BPW), jnp.int32),
            pltpu.VMEM((_NUM_FIELDS, _BPW), jnp.int32),
            pltpu.VMEM((_BPW, 128), jnp.float32),
            pltpu.VMEM((_BPW, 128), jnp.float32),
            pltpu.VMEM((_NUM_FIELDS, 2, 8, _BPW), jnp.float32),
            pltpu.SemaphoreType.DMA,
            pltpu.SemaphoreType.DMA,
        ],
        compiler_params=pltpu.CompilerParams(
            use_tc_tiling_on_sc=True, needs_layout_passes=False
        ),
    )(_gather_body)
    return run(idx_t, offsets, tab)


def kernel(input_x, table, offsets):
    tab = _convert_table(table.T)
    out5 = _fmembedding(input_x.T, offsets, tab)
    return out5.transpose(2, 4, 0, 1, 3).reshape(_BATCH, _NUM_FIELDS, _EMBED_DIM)


# speedup vs baseline: 6.1790x; 1.2950x over previous
import functools

import jax
import jax.numpy as jnp
from jax import lax
from jax.experimental import pallas as pl
from jax.experimental.pallas import tpu as pltpu
from jax.experimental.pallas import tpu_sc as plsc

_NUM_FIELDS = 26
_FIELD_DIM = 38462
_VOCAB = _NUM_FIELDS * _FIELD_DIM
_EMBED_DIM = 16
_BATCH = 4096
_NUM_WORKERS = 32
_BPW = _BATCH // _NUM_WORKERS
_LANES = 16

_CONV_COLS = 4096
_CONV_GRID = -(-_VOCAB // _CONV_COLS)
_CONV_ROWS = _CONV_GRID * (_CONV_COLS // 8)


def _conv_body(x_ref, o_ref):
    x = x_ref[...]
    for s in range(_CONV_COLS // 1024):
        xs = x[:, 1024 * s:1024 * s + 1024]
        stacked = jnp.concatenate(
            [xs[:, 128 * u:128 * u + 128] for u in range(8)], axis=0
        )
        o_ref[pl.ds(128 * s, 128), :] = stacked.T


@jax.jit
def _convert_table(table_t):
    return pl.pallas_call(
        _conv_body,
        grid=(_CONV_GRID,),
        in_specs=[
            pl.BlockSpec((_EMBED_DIM, _CONV_COLS), lambda i: (0, i)),
        ],
        out_specs=pl.BlockSpec((_CONV_COLS // 8, 128), lambda i: (i, 0)),
        compiler_params=pltpu.CompilerParams(
            dimension_semantics=("arbitrary",),
            fuse_transposed_lhs_in_matmul=True,
        ),
        out_shape=jax.ShapeDtypeStruct((_CONV_ROWS, 128), jnp.float32),
    )(table_t)


def _gather_body(idx_hbm, offs_hbm, tab_hbm, out_hbm,
                 idx_v, offs_v, row_v, cb_v, buf0, buf1, slab_v,
                 sem0, sem1):
    wid = lax.axis_index("s") * 2 + lax.axis_index("c")
    base = wid * _BPW

    pltpu.sync_copy(idx_hbm.at[:, pl.ds(base, _BPW)], idx_v)
    pltpu.sync_copy(offs_hbm, offs_v)

    def adjust(f, carry):
        fvec = jnp.full((_LANES,), f, dtype=jnp.int32)
        off = plsc.load_gather(offs_v, [fvec])
        for j in range(_BPW // _LANES):
            sl = pl.ds(j * _LANES, _LANES)
            v = idx_v[f, sl] + off
            row_v[f, sl] = lax.shift_left(
                lax.shift_right_logical(v, 10), 7
            ) + jnp.bitwise_and(v, 127)
            cb_v[f, sl] = lax.shift_left(
                jnp.bitwise_and(lax.shift_right_logical(v, 7), 7), 4
            )
        return carry

    lax.fori_loop(0, _NUM_FIELDS, adjust, 0)

    def fire(f, buf, sem):
        pltpu.async_copy(tab_hbm.at[row_v.at[f]], buf, sem)

    def drain(f, buf, sem):
        pltpu.make_async_copy(tab_hbm.at[row_v.at[f]], buf, sem).wait()

    def extract(f, buf):
        for j in range(_BPW // _LANES):
            rows = jnp.arange(_LANES, dtype=jnp.int32) + (j * _LANES)
            cb = cb_v[f, pl.ds(j * _LANES, _LANES)]
            for e in range(_EMBED_DIM):
                vals = plsc.load_gather(buf, [rows, cb + e])
                slab_v[f, e // 8, e % 8, pl.ds(j * _LANES, _LANES)] = vals

    fire(0, buf0, sem0)
    fire(1, buf1, sem1)

    def step(g, carry):
        f = g * 2
        drain(f, buf0, sem0)
        extract(f, buf0)

        @pl.when(f + 2 < _NUM_FIELDS)
        def _():
            fire(f + 2, buf0, sem0)

        drain(f + 1, buf1, sem1)
        extract(f + 1, buf1)

        @pl.when(f + 3 < _NUM_FIELDS)
        def _():
            fire(f + 3, buf1, sem1)

        return carry

    lax.fori_loop(0, _NUM_FIELDS // 2, step, 0)

    pltpu.sync_copy(slab_v, out_hbm.at[:, :, wid])


@jax.jit
def _fmembedding(idx_t, offsets, tab):
    mesh = plsc.VectorSubcoreMesh(
        core_axis_name="c", subcore_axis_name="s", num_cores=2, num_subcores=16
    )
    run = functools.partial(
        pl.kernel,
        out_type=jax.ShapeDtypeStruct(
            (_NUM_FIELDS, 2, _NUM_WORKERS, 8, _BPW), jnp.float32
        ),
        mesh=mesh,
        scratch_types=[
            pltpu.VMEM((_NUM_FIELDS, _BPW), jnp.int32),
            pltpu.VMEM((_NUM_FIELDS,), jnp.int32),
            pltpu.VMEM((_NUM_FIELDS, _BPW), jnp.int32),
            pltpu.VMEM((_NUM_FIELDS, _BPW), jnp.int32),
            pltpu.VMEM((_BPW, 128), jnp.float32),
            pltpu.VMEM((_BPW, 128), jnp.float32),
            pltpu.VMEM((_NUM_FIELDS, 2, 8, _BPW), jnp.float32),
            pltpu.SemaphoreType.DMA,
            pltpu.SemaphoreType.DMA,
        ],
        compiler_params=pltpu.CompilerParams(
            use_tc_tiling_on_sc=True, needs_layout_passes=False
        ),
    )(_gather_body)
    return run(idx_t, offsets, tab)


def kernel(input_x, table, offsets):
    tab = _convert_table(table.T)
    out5 = _fmembedding(input_x.T, offsets, tab)
    return out5.transpose(2, 4, 0, 1, 3).reshape(_BATCH, _NUM_FIELDS, _EMBED_DIM)


# speedup vs baseline: 10.3812x; 1.6801x over previous
import functools

import jax
import jax.numpy as jnp
from jax import lax
from jax.experimental import pallas as pl
from jax.experimental.pallas import tpu as pltpu
from jax.experimental.pallas import tpu_sc as plsc

_NUM_FIELDS = 26
_FIELD_DIM = 38462
_VOCAB = _NUM_FIELDS * _FIELD_DIM
_EMBED_DIM = 16
_BATCH = 4096
_NUM_WORKERS = 32
_BPW = _BATCH // _NUM_WORKERS
_LANES = 16

_CONV_COLS = 16384
_CONV_GRID = -(-_VOCAB // _CONV_COLS)
_CONV_ROWS = _CONV_GRID * (_CONV_COLS // 8)


def _conv_body(x_ref, o_ref):
    x = x_ref[...]
    for s in range(_CONV_COLS // 1024):
        xs = x[:, 1024 * s:1024 * s + 1024]
        stacked = jnp.concatenate(
            [xs[:, 128 * u:128 * u + 128] for u in range(8)], axis=0
        )
        o_ref[pl.ds(128 * s, 128), :] = stacked.T


@jax.jit
def _convert_table(table_t):
    return pl.pallas_call(
        _conv_body,
        grid=(_CONV_GRID,),
        in_specs=[
            pl.BlockSpec((_EMBED_DIM, _CONV_COLS), lambda i: (0, i)),
        ],
        out_specs=pl.BlockSpec((_CONV_COLS // 8, 128), lambda i: (i, 0)),
        compiler_params=pltpu.CompilerParams(
            dimension_semantics=("arbitrary",),
            fuse_transposed_lhs_in_matmul=True,
        ),
        out_shape=jax.ShapeDtypeStruct((_CONV_ROWS, 128), jnp.float32),
    )(table_t)


def _gather_body(idx_hbm, offs_hbm, tab_hbm, out_hbm,
                 idx_v, offs_v, row_v, cb_v, buf0, buf1, slab_v,
                 sem0, sem1):
    wid = lax.axis_index("s") * 2 + lax.axis_index("c")
    base = wid * _BPW

    pltpu.sync_copy(idx_hbm.at[:, pl.ds(base, _BPW)], idx_v)
    pltpu.sync_copy(offs_hbm, offs_v)

    def adjust(f, carry):
        fvec = jnp.full((_LANES,), f, dtype=jnp.int32)
        off = plsc.load_gather(offs_v, [fvec])
        for j in range(_BPW // _LANES):
            sl = pl.ds(j * _LANES, _LANES)
            v = idx_v[f, sl] + off
            row_v[f, sl] = lax.shift_left(
                lax.shift_right_logical(v, 10), 7
            ) + jnp.bitwise_and(v, 127)
            cb_v[f, sl] = lax.shift_left(
                jnp.bitwise_and(lax.shift_right_logical(v, 7), 7), 4
            )
        return carry

    lax.fori_loop(0, _NUM_FIELDS, adjust, 0)

    def fire(f, buf, sem):
        pltpu.async_copy(tab_hbm.at[row_v.at[f]], buf, sem)

    def drain(f, buf, sem):
        pltpu.make_async_copy(tab_hbm.at[row_v.at[f]], buf, sem).wait()

    def extract(f, buf):
        for j in range(_BPW // _LANES):
            rows = jnp.arange(_LANES, dtype=jnp.int32) + (j * _LANES)
            cb = cb_v[f, pl.ds(j * _LANES, _LANES)]
            for e in range(_EMBED_DIM):
                vals = plsc.load_gather(buf, [rows, cb + e])
                slab_v[f, e // 8, e % 8, pl.ds(j * _LANES, _LANES)] = vals

    fire(0, buf0, sem0)
    fire(1, buf1, sem1)

    def step(g, carry):
        f = g * 2
        drain(f, buf0, sem0)
        extract(f, buf0)

        @pl.when(f + 2 < _NUM_FIELDS)
        def _():
            fire(f + 2, buf0, sem0)

        drain(f + 1, buf1, sem1)
        extract(f + 1, buf1)

        @pl.when(f + 3 < _NUM_FIELDS)
        def _():
            fire(f + 3, buf1, sem1)

        return carry

    lax.fori_loop(0, _NUM_FIELDS // 2, step, 0)

    pltpu.sync_copy(slab_v, out_hbm.at[:, :, wid])


@jax.jit
def _fmembedding(idx_t, offsets, tab):
    mesh = plsc.VectorSubcoreMesh(
        core_axis_name="c", subcore_axis_name="s", num_cores=2, num_subcores=16
    )
    run = functools.partial(
        pl.kernel,
        out_type=jax.ShapeDtypeStruct(
            (_NUM_FIELDS, 2, _NUM_WORKERS, 8, _BPW), jnp.float32
        ),
        mesh=mesh,
        scratch_types=[
            pltpu.VMEM((_NUM_FIELDS, _BPW), jnp.int32),
            pltpu.VMEM((_NUM_FIELDS,), jnp.int32),
            pltpu.VMEM((_NUM_FIELDS, _BPW), jnp.int32),
            pltpu.VMEM((_NUM_FIELDS, _BPW), jnp.int32),
            pltpu.VMEM((_BPW, 128), jnp.float32),
            pltpu.VMEM((_BPW, 128), jnp.float32),
            pltpu.VMEM((_NUM_FIELDS, 2, 8, _BPW), jnp.float32),
            pltpu.SemaphoreType.DMA,
            pltpu.SemaphoreType.DMA,
        ],
        compiler_params=pltpu.CompilerParams(
            use_tc_tiling_on_sc=True, needs_layout_passes=False
        ),
    )(_gather_body)
    return run(idx_t, offsets, tab)


def kernel(input_x, table, offsets):
    tab = _convert_table(table.T)
    out5 = _fmembedding(input_x.T, offsets, tab)
    return out5.transpose(2, 4, 0, 1, 3).reshape(_BATCH, _NUM_FIELDS, _EMBED_DIM)


# speedup vs baseline: 13.0885x; 1.2608x over previous
import functools

import jax
import jax.numpy as jnp
from jax import lax
from jax.experimental import pallas as pl
from jax.experimental.pallas import tpu as pltpu
from jax.experimental.pallas import tpu_sc as plsc

_NUM_FIELDS = 26
_FIELD_DIM = 38462
_VOCAB = _NUM_FIELDS * _FIELD_DIM
_EMBED_DIM = 16
_BATCH = 4096
_NUM_WORKERS = 32
_BPW = _BATCH // _NUM_WORKERS
_LANES = 16

_CONV_COLS = 65536
_CONV_GRID = -(-_VOCAB // _CONV_COLS)
_CONV_ROWS = _CONV_GRID * (_CONV_COLS // 8)


def _conv_body(x_ref, o_ref):
    x = x_ref[...]
    for s in range(_CONV_COLS // 1024):
        xs = x[:, 1024 * s:1024 * s + 1024]
        stacked = jnp.concatenate(
            [xs[:, 128 * u:128 * u + 128] for u in range(8)], axis=0
        )
        o_ref[pl.ds(128 * s, 128), :] = stacked.T


@jax.jit
def _convert_table(table_t):
    return pl.pallas_call(
        _conv_body,
        grid=(_CONV_GRID,),
        in_specs=[
            pl.BlockSpec((_EMBED_DIM, _CONV_COLS), lambda i: (0, i)),
        ],
        out_specs=pl.BlockSpec((_CONV_COLS // 8, 128), lambda i: (i, 0)),
        compiler_params=pltpu.CompilerParams(
            dimension_semantics=("arbitrary",),
            fuse_transposed_lhs_in_matmul=True,
        ),
        out_shape=jax.ShapeDtypeStruct((_CONV_ROWS, 128), jnp.float32),
    )(table_t)


def _gather_body(idx_hbm, offs_hbm, tab_hbm, out_hbm,
                 idx_v, offs_v, row_v, cb_v, buf0, buf1, slab_v,
                 sem0, sem1):
    wid = lax.axis_index("s") * 2 + lax.axis_index("c")
    base = wid * _BPW

    pltpu.sync_copy(idx_hbm.at[:, pl.ds(base, _BPW)], idx_v)
    pltpu.sync_copy(offs_hbm, offs_v)

    def adjust(f, carry):
        fvec = jnp.full((_LANES,), f, dtype=jnp.int32)
        off = plsc.load_gather(offs_v, [fvec])
        for j in range(_BPW // _LANES):
            sl = pl.ds(j * _LANES, _LANES)
            v = idx_v[f, sl] + off
            row_v[f, sl] = lax.shift_left(
                lax.shift_right_logical(v, 10), 7
            ) + jnp.bitwise_and(v, 127)
            cb_v[f, sl] = lax.shift_left(
                jnp.bitwise_and(lax.shift_right_logical(v, 7), 7), 4
            )
        return carry

    lax.fori_loop(0, _NUM_FIELDS, adjust, 0)

    def fire(f, buf, sem):
        pltpu.async_copy(tab_hbm.at[row_v.at[f]], buf, sem)

    def drain(f, buf, sem):
        pltpu.make_async_copy(tab_hbm.at[row_v.at[f]], buf, sem).wait()

    def extract(f, buf):
        for j in range(_BPW // _LANES):
            rows = jnp.arange(_LANES, dtype=jnp.int32) + (j * _LANES)
            cb = cb_v[f, pl.ds(j * _LANES, _LANES)]
            for e in range(_EMBED_DIM):
                vals = plsc.load_gather(buf, [rows, cb + e])
                slab_v[f, e // 8, e % 8, pl.ds(j * _LANES, _LANES)] = vals

    fire(0, buf0, sem0)
    fire(1, buf1, sem1)

    def step(g, carry):
        f = g * 2
        drain(f, buf0, sem0)
        extract(f, buf0)

        @pl.when(f + 2 < _NUM_FIELDS)
        def _():
            fire(f + 2, buf0, sem0)

        drain(f + 1, buf1, sem1)
        extract(f + 1, buf1)

        @pl.when(f + 3 < _NUM_FIELDS)
        def _():
            fire(f + 3, buf1, sem1)

        return carry

    lax.fori_loop(0, _NUM_FIELDS // 2, step, 0)

    pltpu.sync_copy(slab_v, out_hbm.at[:, :, wid])


@jax.jit
def _fmembedding(idx_t, offsets, tab):
    mesh = plsc.VectorSubcoreMesh(
        core_axis_name="c", subcore_axis_name="s", num_cores=2, num_subcores=16
    )
    run = functools.partial(
        pl.kernel,
        out_type=jax.ShapeDtypeStruct(
            (_NUM_FIELDS, 2, _NUM_WORKERS, 8, _BPW), jnp.float32
        ),
        mesh=mesh,
        scratch_types=[
            pltpu.VMEM((_NUM_FIELDS, _BPW), jnp.int32),
            pltpu.VMEM((_NUM_FIELDS,), jnp.int32),
            pltpu.VMEM((_NUM_FIELDS, _BPW), jnp.int32),
            pltpu.VMEM((_NUM_FIELDS, _BPW), jnp.int32),
            pltpu.VMEM((_BPW, 128), jnp.float32),
            pltpu.VMEM((_BPW, 128), jnp.float32),
            pltpu.VMEM((_NUM_FIELDS, 2, 8, _BPW), jnp.float32),
            pltpu.SemaphoreType.DMA,
            pltpu.SemaphoreType.DMA,
        ],
        compiler_params=pltpu.CompilerParams(
            use_tc_tiling_on_sc=True, needs_layout_passes=False
        ),
    )(_gather_body)
    return run(idx_t, offsets, tab)


def kernel(input_x, table, offsets):
    tab = _convert_table(table.T)
    out5 = _fmembedding(input_x.T, offsets, tab)
    return out5.transpose(2, 4, 0, 1, 3).reshape(_BATCH, _NUM_FIELDS, _EMBED_DIM)


# speedup vs baseline: 13.2269x; 1.0106x over previous
import functools

import jax
import jax.numpy as jnp
from jax import lax
from jax.experimental import pallas as pl
from jax.experimental.pallas import tpu as pltpu
from jax.experimental.pallas import tpu_sc as plsc

_NUM_FIELDS = 26
_FIELD_DIM = 38462
_VOCAB = _NUM_FIELDS * _FIELD_DIM
_EMBED_DIM = 16
_BATCH = 4096
_NUM_WORKERS = 32
_BPW = _BATCH // _NUM_WORKERS
_LANES = 16

_CONV_COLS = 131072
_CONV_GRID = -(-_VOCAB // _CONV_COLS)
_CONV_ROWS = _CONV_GRID * (_CONV_COLS // 8)


def _conv_body(x_ref, o_ref):
    x = x_ref[...]
    for s in range(_CONV_COLS // 1024):
        xs = x[:, 1024 * s:1024 * s + 1024]
        stacked = jnp.concatenate(
            [xs[:, 128 * u:128 * u + 128] for u in range(8)], axis=0
        )
        o_ref[pl.ds(128 * s, 128), :] = stacked.T


@jax.jit
def _convert_table(table_t):
    return pl.pallas_call(
        _conv_body,
        grid=(_CONV_GRID,),
        in_specs=[
            pl.BlockSpec((_EMBED_DIM, _CONV_COLS), lambda i: (0, i)),
        ],
        out_specs=pl.BlockSpec((_CONV_COLS // 8, 128), lambda i: (i, 0)),
        compiler_params=pltpu.CompilerParams(
            dimension_semantics=("arbitrary",),
            fuse_transposed_lhs_in_matmul=True,
        ),
        out_shape=jax.ShapeDtypeStruct((_CONV_ROWS, 128), jnp.float32),
    )(table_t)


def _gather_body(idx_hbm, offs_hbm, tab_hbm, out_hbm,
                 idx_v, offs_v, row_v, cb_v, buf0, buf1, slab_v,
                 sem0, sem1):
    wid = lax.axis_index("s") * 2 + lax.axis_index("c")
    base = wid * _BPW

    pltpu.sync_copy(idx_hbm.at[:, pl.ds(base, _BPW)], idx_v)
    pltpu.sync_copy(offs_hbm, offs_v)

    def adjust(f, carry):
        fvec = jnp.full((_LANES,), f, dtype=jnp.int32)
        off = plsc.load_gather(offs_v, [fvec])
        for j in range(_BPW // _LANES):
            sl = pl.ds(j * _LANES, _LANES)
            v = idx_v[f, sl] + off
            row_v[f, sl] = lax.shift_left(
                lax.shift_right_logical(v, 10), 7
            ) + jnp.bitwise_and(v, 127)
            cb_v[f, sl] = lax.shift_left(
                jnp.bitwise_and(lax.shift_right_logical(v, 7), 7), 4
            )
        return carry

    lax.fori_loop(0, _NUM_FIELDS, adjust, 0)

    def fire(f, buf, sem):
        pltpu.async_copy(tab_hbm.at[row_v.at[f]], buf, sem)

    def drain(f, buf, sem):
        pltpu.make_async_copy(tab_hbm.at[row_v.at[f]], buf, sem).wait()

    def extract(f, buf):
        for j in range(_BPW // _LANES):
            rows = jnp.arange(_LANES, dtype=jnp.int32) + (j * _LANES)
            cb = cb_v[f, pl.ds(j * _LANES, _LANES)]
            for e in range(_EMBED_DIM):
                vals = plsc.load_gather(buf, [rows, cb + e])
                slab_v[f, e // 8, e % 8, pl.ds(j * _LANES, _LANES)] = vals

    fire(0, buf0, sem0)
    fire(1, buf1, sem1)

    def step(g, carry):
        f = g * 2
        drain(f, buf0, sem0)
        extract(f, buf0)

        @pl.when(f + 2 < _NUM_FIELDS)
        def _():
            fire(f + 2, buf0, sem0)

        drain(f + 1, buf1, sem1)
        extract(f + 1, buf1)

        @pl.when(f + 3 < _NUM_FIELDS)
        def _():
            fire(f + 3, buf1, sem1)

        return carry

    lax.fori_loop(0, _NUM_FIELDS // 2, step, 0)

    pltpu.sync_copy(slab_v, out_hbm.at[:, :, wid])


@jax.jit
def _fmembedding(idx_t, offsets, tab):
    mesh = plsc.VectorSubcoreMesh(
        core_axis_name="c", subcore_axis_name="s", num_cores=2, num_subcores=16
    )
    run = functools.partial(
        pl.kernel,
        out_type=jax.ShapeDtypeStruct(
            (_NUM_FIELDS, 2, _NUM_WORKERS, 8, _BPW), jnp.float32
        ),
        mesh=mesh,
        scratch_types=[
            pltpu.VMEM((_NUM_FIELDS, _BPW), jnp.int32),
            pltpu.VMEM((_NUM_FIELDS,), jnp.int32),
            pltpu.VMEM((_NUM_FIELDS, _BPW), jnp.int32),
            pltpu.VMEM((_NUM_FIELDS, _BPW), jnp.int32),
            pltpu.VMEM((_BPW, 128), jnp.float32),
            pltpu.VMEM((_BPW, 128), jnp.float32),
            pltpu.VMEM((_NUM_FIELDS, 2, 8, _BPW), jnp.float32),
            pltpu.SemaphoreType.DMA,
            pltpu.SemaphoreType.DMA,
        ],
        compiler_params=pltpu.CompilerParams(
            use_tc_tiling_on_sc=True, needs_layout_passes=False
        ),
    )(_gather_body)
    return run(idx_t, offsets, tab)


def kernel(input_x, table, offsets):
    tab = _convert_table(table.T)
    out5 = _fmembedding(input_x.T, offsets, tab)
    return out5.transpose(2, 4, 0, 1, 3).reshape(_BATCH, _NUM_FIELDS, _EMBED_DIM)
